# baseline (speedup 1.0000x reference)
import functools

import jax
import jax.numpy as jnp
from jax import lax
from jax.experimental import pallas as pl
from jax.experimental.pallas import tpu as pltpu
from jax.experimental.pallas import tpu_sc as plsc

_VOCAB = 100000
_EMBED = 64
_BATCH = 1024

_BN = 3072
_NFULL = _VOCAB // _BN
_SHIFT = _BN - (_VOCAB - _NFULL * _BN)


_PB = 2176
_HALF = 50048


def _repack_block(e_ref, o_ref, out_ref):
    out_ref[...] = jnp.concatenate([e_ref[...].T, o_ref[...].T], axis=1)


def _repack(Tt, interpret=False):
    _nhb = _HALF // _PB
    return pl.pallas_call(
        _repack_block,
        grid=(_HALF // _PB,),
        in_specs=[
            pl.BlockSpec((_EMBED, _PB), lambda p: (0, p)),
            pl.BlockSpec((_EMBED, _PB), lambda p: (0, _nhb + p)),
        ],
        out_specs=pl.BlockSpec((_PB, 2 * _EMBED), lambda p: (p, 0)),
        out_shape=jax.ShapeDtypeStruct((_HALF, 2 * _EMBED), jnp.float32),
        compiler_params=pltpu.CompilerParams(
            dimension_semantics=("arbitrary",),
            vmem_limit_bytes=100 * 1024 * 1024,
        ),
        interpret=interpret,
    )(Tt, Tt)


def _gather_sc(xh, emb2):
    info = plsc.get_sparse_core_info()
    nc, ns = info.num_cores, info.num_subcores
    nw = nc * ns
    b_per_w = _BATCH // nw
    mesh = plsc.VectorSubcoreMesh(core_axis_name="c", subcore_axis_name="s")

    @functools.partial(
        pl.kernel,
        mesh=mesh,
        out_type=jax.ShapeDtypeStruct((_BATCH, 2 * _EMBED), jnp.float32),
        scratch_types=[
            pltpu.VMEM((b_per_w,), jnp.int32),
            pltpu.VMEM((b_per_w, 2 * _EMBED), jnp.float32),
            pltpu.SemaphoreType.DMA,
        ],
    )
    def k(table_hbm, idx_hbm, out_hbm, idx_v, rows_v, sem):
        wid = lax.axis_index("s") * nc + lax.axis_index("c")
        base = wid * b_per_w
        pltpu.sync_copy(idx_hbm.at[pl.ds(base, b_per_w)], idx_v)
        pltpu.async_copy(table_hbm.at[idx_v], rows_v, sem).wait()
        pltpu.sync_copy(rows_v, out_hbm.at[pl.ds(base, b_per_w)])

    return k(emb2, xh)


def _matmul_block_t(h2_ref, p_ref, wt_ref, b_ref, out_ref, h_ref):
    j = pl.program_id(0)

    @pl.when(j == 0)
    def _():
        h_ref[...] = jnp.where(
            p_ref[...] > 0,
            h2_ref[:, _EMBED:],
            h2_ref[:, :_EMBED],
        )

    h = h_ref[...]

    def dot_at(off):
        return lax.dot_general(
            wt_ref[:, pl.ds(off, _BN)],
            h,
            (((0,), (1,)), ((), ())),
            preferred_element_type=jnp.float32,
        )

    @pl.when(j < _NFULL)
    def _():
        out_ref[...] = dot_at(j * _BN) + b_ref[...][:, None]

    @pl.when(j == _NFULL)
    def _():
        d = dot_at(_VOCAB - _BN)
        rolled = jnp.concatenate([d[_SHIFT:], d[:_SHIFT]], axis=0)
        out_ref[...] = rolled + b_ref[...][:, None]


def _project_t(h2, parity, Wt, b, interpret=False):
    return pl.pallas_call(
        _matmul_block_t,
        grid=(pl.cdiv(_VOCAB, _BN),),
        in_specs=[
            pl.BlockSpec((_BATCH, 2 * _EMBED), lambda j: (0, 0)),
            pl.BlockSpec((_BATCH, 1), lambda j: (0, 0)),
            pl.BlockSpec((_EMBED, _VOCAB), lambda j: (0, 0)),
            pl.BlockSpec((_BN,), lambda j: (j,)),
        ],
        out_specs=pl.BlockSpec((_BN, _BATCH), lambda j: (j, 0)),
        out_shape=jax.ShapeDtypeStruct((_VOCAB, _BATCH), jnp.float32),
        compiler_params=pltpu.CompilerParams(
            dimension_semantics=("arbitrary",),
            vmem_limit_bytes=100 * 1024 * 1024,
        ),
        scratch_shapes=[pltpu.VMEM((_BATCH, _EMBED), jnp.float32)],
        interpret=interpret,
    )(h2, parity, Wt, b)


def kernel(x, emb_table, W, b):
    xi = x.astype(jnp.int32)
    emb2 = _repack(emb_table.T)
    row = jnp.where(xi < _HALF, xi, xi - _HALF)
    h2 = _gather_sc(row, emb2)
    parity = (xi >= _HALF).astype(jnp.float32).reshape(_BATCH, 1)
    out_t = _project_t(h2, parity, W.T, b)
    return out_t.T


# speedup vs baseline: 1.0224x; 1.0224x over previous
import functools

import jax
import jax.numpy as jnp
from jax import lax
from jax.experimental import pallas as pl
from jax.experimental.pallas import tpu as pltpu
from jax.experimental.pallas import tpu_sc as plsc

_VOCAB = 100000
_EMBED = 64
_BATCH = 1024

_BN = 2048
_NFULL = _VOCAB // _BN
_SHIFT = _BN - (_VOCAB - _NFULL * _BN)


_PB = 2944
_HALF = 50048


def _repack_block(e_ref, o_ref, out_ref):
    out_ref[...] = jnp.concatenate([e_ref[...].T, o_ref[...].T], axis=1)


def _repack(Tt, interpret=False):
    _nhb = _HALF // _PB
    return pl.pallas_call(
        _repack_block,
        grid=(_HALF // _PB,),
        in_specs=[
            pl.BlockSpec((_EMBED, _PB), lambda p: (0, p)),
            pl.BlockSpec((_EMBED, _PB), lambda p: (0, _nhb + p)),
        ],
        out_specs=pl.BlockSpec((_PB, 2 * _EMBED), lambda p: (p, 0)),
        out_shape=jax.ShapeDtypeStruct((_HALF, 2 * _EMBED), jnp.float32),
        compiler_params=pltpu.CompilerParams(
            dimension_semantics=("arbitrary",),
            vmem_limit_bytes=100 * 1024 * 1024,
        ),
        interpret=interpret,
    )(Tt, Tt)


def _gather_sc(xh, emb2):
    info = plsc.get_sparse_core_info()
    nc, ns = info.num_cores, info.num_subcores
    nw = nc * ns
    b_per_w = _BATCH // nw
    mesh = plsc.VectorSubcoreMesh(core_axis_name="c", subcore_axis_name="s")

    @functools.partial(
        pl.kernel,
        mesh=mesh,
        out_type=jax.ShapeDtypeStruct((_BATCH, 2 * _EMBED), jnp.float32),
        scratch_types=[
            pltpu.VMEM((b_per_w,), jnp.int32),
            pltpu.VMEM((b_per_w, 2 * _EMBED), jnp.float32),
            pltpu.SemaphoreType.DMA,
        ],
    )
    def k(table_hbm, idx_hbm, out_hbm, idx_v, rows_v, sem):
        wid = lax.axis_index("s") * nc + lax.axis_index("c")
        base = wid * b_per_w
        pltpu.sync_copy(idx_hbm.at[pl.ds(base, b_per_w)], idx_v)
        pltpu.async_copy(table_hbm.at[idx_v], rows_v, sem).wait()
        pltpu.sync_copy(rows_v, out_hbm.at[pl.ds(base, b_per_w)])

    return k(emb2, xh)


def _matmul_block_t(h2_ref, p_ref, wt_ref, b_ref, out_ref, h_ref):
    j = pl.program_id(0)

    @pl.when(j == 0)
    def _():
        h_ref[...] = jnp.where(
            p_ref[...] > 0,
            h2_ref[:, _EMBED:],
            h2_ref[:, :_EMBED],
        )

    h = h_ref[...]

    def dot_at(off):
        return lax.dot_general(
            wt_ref[:, pl.ds(off, _BN)],
            h,
            (((0,), (1,)), ((), ())),
            preferred_element_type=jnp.float32,
        )

    @pl.when(j < _NFULL)
    def _():
        out_ref[...] = dot_at(j * _BN) + b_ref[...][:, None]

    @pl.when(j == _NFULL)
    def _():
        d = dot_at(_VOCAB - _BN)
        rolled = jnp.concatenate([d[_SHIFT:], d[:_SHIFT]], axis=0)
        out_ref[...] = rolled + b_ref[...][:, None]


def _project_t(h2, parity, Wt, b, interpret=False):
    return pl.pallas_call(
        _matmul_block_t,
        grid=(pl.cdiv(_VOCAB, _BN),),
        in_specs=[
            pl.BlockSpec((_BATCH, 2 * _EMBED), lambda j: (0, 0)),
            pl.BlockSpec((_BATCH, 1), lambda j: (0, 0)),
            pl.BlockSpec((_EMBED, _VOCAB), lambda j: (0, 0)),
            pl.BlockSpec((_BN,), lambda j: (j,)),
        ],
        out_specs=pl.BlockSpec((_BN, _BATCH), lambda j: (j, 0)),
        out_shape=jax.ShapeDtypeStruct((_VOCAB, _BATCH), jnp.float32),
        compiler_params=pltpu.CompilerParams(
            dimension_semantics=("arbitrary",),
            vmem_limit_bytes=100 * 1024 * 1024,
        ),
        scratch_shapes=[pltpu.VMEM((_BATCH, _EMBED), jnp.float32)],
        interpret=interpret,
    )(h2, parity, Wt, b)


def kernel(x, emb_table, W, b):
    xi = x.astype(jnp.int32)
    emb2 = _repack(emb_table.T)
    row = jnp.where(xi < _HALF, xi, xi - _HALF)
    h2 = _gather_sc(row, emb2)
    parity = (xi >= _HALF).astype(jnp.float32).reshape(_BATCH, 1)
    out_t = _project_t(h2, parity, W.T, b)
    return out_t.T
